# Initial kernel scaffold; baseline (speedup 1.0000x reference)
import functools

import jax
import jax.numpy as jnp
from jax import lax
from jax.experimental import pallas as pl
from jax.experimental.pallas import tpu as pltpu

try:
    from jax.experimental.pallas import tpu_sc as plsc
    _HAS_SC = True
except ImportError:
    _HAS_SC = False

_K1 = 15
_KN = 20
_M = 1536
_N = 4096
_T3 = _M * (_K1 - 1)
_TARGET = 512
_HI = lax.Precision.HIGHEST


def _sq3(g):
    p = g * g
    return (p[:, 0] + p[:, 2]) + p[:, 1]


def _sum64_rows(x):
    y = x[:, 0:8]
    for i in range(1, 8):
        y = y + x[:, 8 * i:8 * (i + 1)]
    y = y[:, 0:4] + y[:, 4:8]
    y = y[:, 0:2] + y[:, 2:4]
    return y[:, 0:1] + y[:, 1:2]


def _sum20_rows(x):
    x = jnp.concatenate([x, jnp.zeros((x.shape[0], 12), jnp.float32)], axis=1)
    n = 32
    while n > 1:
        n //= 2
        x = x[:, :n] + x[:, n:]
    return x


def _rowsum1536(s):
    acc = s[:, 0:128]
    for t in range(1, 12):
        acc = acc + s[:, 128 * t:128 * (t + 1)]
    y = acc[:, 0:8]
    for i in range(1, 16):
        y = y + acc[:, 8 * i:8 * (i + 1)]
    y = y[:, 0:4] + y[:, 4:8]
    y = y[:, 0:2] + y[:, 2:4]
    return y[:, 0:1] + y[:, 1:2]


def _topk_rows(nd, k):
    R, C = nd.shape
    cols = lax.broadcasted_iota(jnp.int32, (R, C), 1)
    idxs = []
    for _ in range(k):
        m = jnp.max(nd, axis=1, keepdims=True)
        idx = jnp.min(jnp.where(nd == m, cols, C), axis=1)
        idxs.append(idx)
        nd = jnp.where(cols == idx[:, None], -jnp.inf, nd)
    return jnp.stack(idxs, axis=1)


def _score_kernel(adj_ref, nodes_ref, W1_ref, w2_ref, out_ref):
    h = jax.nn.relu(jnp.dot(nodes_ref[...], W1_ref[...]))
    aggT = lax.dot_general(h, adj_ref[...], (((0,), (1,)), ((), ())))
    out_ref[...] = jnp.dot(aggT.T, w2_ref[...])


def _select_kernel(score_ref, g_ref, nodes_ref, iota_ref, sel_ref, gen_ref):
    s = score_ref[...]
    sh = s - jnp.max(s)
    v = (sh - jnp.log(jnp.sum(jnp.exp(sh))) + g_ref[...])[0]
    jcols = iota_ref[...][0]
    ranks = []
    for c in range(8):
        vi = v[c * 512:(c + 1) * 512]
        ii = jcols[c * 512:(c + 1) * 512]
        gt = v[None, :] > vi[:, None]
        eqlt = (v[None, :] == vi[:, None]) & (jcols[None, :] < ii[:, None])
        ranks.append(jnp.sum((gt | eqlt).astype(jnp.int32), axis=1))
    rank = jnp.concatenate(ranks)
    nodes = nodes_ref[...]
    for c in range(3):
        r = jcols[0:512] + c * 512
        onehot = rank[None, :] == r[:, None]
        sel_ref[c * 512:(c + 1) * 512, :] = jnp.sum(
            jnp.where(onehot, jcols[None, :], 0), axis=1, keepdims=True)
        gen_ref[c * 512:(c + 1) * 512, :] = jnp.dot(
            onehot.astype(jnp.float32), nodes, precision=_HI)


def _knn1_kernel(chunk_ref, gen_ref, nbr_ref):
    i = pl.program_id(0)
    chunk = chunk_ref[...]
    g = gen_ref[...]
    d = _sq3(chunk)[:, None] + _sq3(g)[None, :] - 2.0 * lax.dot_general(
        chunk, g, (((1,), (1,)), ((), ())))
    rows = lax.broadcasted_iota(jnp.int32, (512, _M), 0) + i * 512
    cols = lax.broadcasted_iota(jnp.int32, (512, _M), 1)
    nd = -jnp.where(rows == cols, jnp.inf, d)
    nbr_ref[...] = _topk_rows(nd, _K1)


def _devconv_kernel(nbr_ref, genb_ref, gen_ref, Wd_ref, f_ref):
    nbrf = nbr_ref[...].astype(jnp.float32).reshape(256 * _K1, 1)
    cols = lax.broadcasted_iota(jnp.float32, (256 * _K1, _M), 1)
    onehot = (nbrf == cols).astype(jnp.float32)
    gnbr = jnp.dot(onehot, gen_ref[...], precision=_HI).reshape(256, _K1, 3)
    diff = gnbr - genb_ref[...][:, None, :]
    z = jax.nn.relu(jnp.dot(diff.reshape(256 * _K1, 3), Wd_ref[...]))
    z = z.reshape(256, _K1, 64)
    se = z[:, 0, :]
    for i in range(1, _K1):
        se = jnp.maximum(se, z[:, i, :])
    f_ref[...] = jax.nn.sigmoid(_sum64_rows(se) / 64.0)


def _rowsum_kernel(nbr_ref, f_ref, rs_ref):
    nbr = nbr_ref[...]
    cols = lax.broadcasted_iota(jnp.int32, (_M, _M), 1)
    A = jnp.zeros((_M, _M), jnp.float32)
    for k in range(_K1):
        A = jnp.maximum(A, (nbr[:, k][:, None] == cols).astype(jnp.float32))
    A = jnp.maximum(A, A.T)
    fv = f_ref[...][:, 0]
    S = A * (fv[:, None] * fv[None, :])
    rs_ref[...] = _rowsum1536(S)


def _packed_kernel(nbr_ref, centerb_ref, table_ref, packed_ref):
    nbr_b = nbr_ref[...]
    nbrf = nbr_b.astype(jnp.float32).reshape(256 * _K1, 1)
    cols = lax.broadcasted_iota(jnp.float32, (256 * _K1, _M), 1)
    onehot = (nbrf == cols).astype(jnp.float32)
    gath = jnp.dot(onehot, table_ref[...], precision=_HI).reshape(256, _K1, 20)
    gnbr = gath[:, :, 0:3]
    f_nbr = gath[:, :, 3]
    rs_nbr = gath[:, :, 4]
    nbr_nbr = gath[:, :, 5:20]
    center = centerb_ref[...]
    v0 = center[:, 0:3]
    f_i = center[:, 3]
    rs_i = center[:, 4]
    pieces = []
    for c in range(_K1 - 1):
        jf = nbr_b[:, c].astype(jnp.float32)
        lf = nbr_b[:, c + 1].astype(jnp.float32)
        f_j, f_l = f_nbr[:, c], f_nbr[:, c + 1]
        rs_j, rs_l = rs_nbr[:, c], rs_nbr[:, c + 1]
        a_jl = (jnp.max((nbr_nbr[:, c, :] == lf[:, None]).astype(jnp.float32), axis=1)
                + jnp.max((nbr_nbr[:, c + 1, :] == jf[:, None]).astype(jnp.float32), axis=1)) > 0.0
        p_ij = 0.5 * ((f_i * f_j) / (rs_i + 1e-9) + (f_j * f_i) / (rs_j + 1e-9))
        p_il = 0.5 * ((f_i * f_l) / (rs_i + 1e-9) + (f_l * f_i) / (rs_l + 1e-9))
        p_jl_val = 0.5 * ((f_j * f_l) / (rs_j + 1e-9) + (f_l * f_j) / (rs_l + 1e-9))
        p_jl = jnp.where(a_jl, p_jl_val, 0.0)
        p_init = (p_ij * p_jl) * p_il
        v1 = gnbr[:, c, :]
        v2 = gnbr[:, c + 1, :]
        bary = ((v0 + v1) + v2) / 3.0
        row = jnp.concatenate(
            [bary, v0, v1, v2, p_init[:, None],
             jnp.zeros((256, 3), jnp.float32)], axis=1)
        pieces.append(row)
    blk = jnp.stack(pieces, axis=1)
    packed_ref[...] = blk.reshape(256 * (_K1 - 1), 16)


def _knn2_kernel(chunk_ref, bary_ref, neigh_ref):
    i = pl.program_id(0)
    chunk = chunk_ref[...]
    b = bary_ref[...]
    d = _sq3(chunk)[:, None] + _sq3(b)[None, :] - 2.0 * lax.dot_general(
        chunk, b, (((1,), (1,)), ((), ())))
    rows = lax.broadcasted_iota(jnp.int32, (128, _T3), 0) + i * 128
    cols = lax.broadcasted_iota(jnp.int32, (128, _T3), 1)
    nd = -jnp.where(rows == cols, jnp.inf, d)
    neigh_ref[...] = _topk_rows(nd, _KN)


def _make_sc_gather():
    info = plsc.get_sparse_core_info()
    NW = info.num_cores * info.num_subcores
    B = _T3 * _KN
    per_w = B // NW
    chunk = 3360
    mesh = plsc.VectorSubcoreMesh(core_axis_name="c", subcore_axis_name="s")

    @functools.partial(
        pl.kernel, mesh=mesh,
        out_type=jax.ShapeDtypeStruct((B, 16), jnp.float32),
        scratch_types=[
            pltpu.VMEM((chunk,), jnp.int32),
            pltpu.VMEM((chunk, 16), jnp.float32),
            pltpu.SemaphoreType.DMA,
        ],
    )
    def sc_gather(packed_hbm, idx_hbm, out_hbm, idx_v, rows_v, sem):
        wid = lax.axis_index("s") * info.num_cores + lax.axis_index("c")
        base = wid * per_w
        for k in range(per_w // chunk):
            off = base + k * chunk
            pltpu.sync_copy(idx_hbm.at[pl.ds(off, chunk)], idx_v)
            pltpu.async_copy(packed_hbm.at[idx_v], rows_v, sem).wait()
            pltpu.sync_copy(rows_v, out_hbm.at[pl.ds(off, chunk)])

    return sc_gather


def _mlp_kernel(exp_ref, packedb_ref, Wm_ref, wv_ref, out_ref):
    nb = exp_ref[...].reshape(1024, _KN, 16)
    center = packedb_ref[...]
    d_bary = nb[:, :, 0:3] - center[:, None, 0:3]
    d_tri = nb[:, :, 3:12] - center[:, None, 3:12]
    rm = jnp.concatenate([d_bary, d_tri], axis=2)
    hm = jax.nn.relu(jnp.dot(rm.reshape(1024 * _KN, 12), Wm_ref[...]))
    hm = hm.reshape(1024, _KN, 128)
    pn = nb[:, :, 12]
    w3 = hm * pn[:, :, None]
    acc = w3[:, 0, :]
    for i in range(1, _KN):
        acc = acc + w3[:, i, :]
    pooled = acc / (_sum20_rows(pn) + 1e-9)
    s = jnp.dot(pooled, wv_ref[...])
    out_ref[...] = jax.nn.sigmoid(s) * center[:, 12:13]


def _rank_kernel(fsb_ref, fs_ref, iota_ref, rank_ref):
    i = pl.program_id(0)
    fs = fs_ref[...][0]
    cols = iota_ref[...][0]
    vi = fsb_ref[...][:, 0]
    ii = cols[0:256] + i * 256
    gt = fs[None, :] > vi[:, None]
    eqlt = (fs[None, :] == vi[:, None]) & (cols[None, :] < ii[:, None])
    rank_ref[...] = jnp.sum((gt | eqlt).astype(jnp.int32), axis=1, keepdims=True)


def _gather_out_kernel(rank_ref, tri_ref, iota_ref, out_ref):
    i = pl.program_id(0)
    r = iota_ref[...][0][0:256] + i * 256
    rank = rank_ref[...][:, 0]
    onehot = (rank[None, :] == r[:, None]).astype(jnp.float32)
    out_ref[...] = jnp.dot(onehot, tri_ref[...], precision=_HI)


def kernel(original_graph_nodes, original_graph_adjacency_matrix, W_gnn1,
           w_gnn2, W_dev, W_mlp1, w_mlp2, target_number_triangles):
    nodes = original_graph_nodes
    adj = original_graph_adjacency_matrix

    u = jax.random.uniform(jax.random.key(42), (_N,), dtype=jnp.float32)
    g = -jnp.log(-jnp.log(u + 1e-9) + 1e-9)

    score = pl.pallas_call(
        _score_kernel, grid=(8,),
        in_specs=[pl.BlockSpec((512, _N), lambda i: (i, 0)),
                  pl.BlockSpec((_N, 3), lambda i: (0, 0)),
                  pl.BlockSpec((3, 64), lambda i: (0, 0)),
                  pl.BlockSpec((64, 1), lambda i: (0, 0))],
        out_specs=pl.BlockSpec((512, 1), lambda i: (i, 0)),
        out_shape=jax.ShapeDtypeStruct((_N, 1), jnp.float32),
    )(adj, nodes, W_gnn1, w_gnn2)

    iota_n = lax.iota(jnp.int32, _N).reshape(1, _N)
    sel, gen = pl.pallas_call(
        _select_kernel,
        out_shape=(jax.ShapeDtypeStruct((_M, 1), jnp.int32),
                   jax.ShapeDtypeStruct((_M, 3), jnp.float32)),
    )(score.reshape(1, _N), g.reshape(1, _N), nodes, iota_n)
    del sel

    nbr = pl.pallas_call(
        _knn1_kernel, grid=(3,),
        in_specs=[pl.BlockSpec((512, 3), lambda i: (i, 0)),
                  pl.BlockSpec((_M, 3), lambda i: (0, 0))],
        out_specs=pl.BlockSpec((512, _K1), lambda i: (i, 0)),
        out_shape=jax.ShapeDtypeStruct((_M, _K1), jnp.int32),
    )(gen, gen)

    f = pl.pallas_call(
        _devconv_kernel, grid=(6,),
        in_specs=[pl.BlockSpec((256, _K1), lambda i: (i, 0)),
                  pl.BlockSpec((256, 3), lambda i: (i, 0)),
                  pl.BlockSpec((_M, 3), lambda i: (0, 0)),
                  pl.BlockSpec((3, 64), lambda i: (0, 0))],
        out_specs=pl.BlockSpec((256, 1), lambda i: (i, 0)),
        out_shape=jax.ShapeDtypeStruct((_M, 1), jnp.float32),
    )(nbr, gen, gen, W_dev)

    rs = pl.pallas_call(
        _rowsum_kernel,
        out_shape=jax.ShapeDtypeStruct((_M, 1), jnp.float32),
    )(nbr, f)

    table = jnp.concatenate([gen, f, rs, nbr.astype(jnp.float32)], axis=1)
    centerb = jnp.concatenate([gen, f, rs], axis=1)
    packed = pl.pallas_call(
        _packed_kernel, grid=(6,),
        in_specs=[pl.BlockSpec((256, _K1), lambda i: (i, 0)),
                  pl.BlockSpec((256, 5), lambda i: (i, 0)),
                  pl.BlockSpec((_M, 20), lambda i: (0, 0))],
        out_specs=pl.BlockSpec((256 * (_K1 - 1), 16), lambda i: (i, 0)),
        out_shape=jax.ShapeDtypeStruct((_T3, 16), jnp.float32),
    )(nbr, centerb, table)

    bary = packed[:, 0:3]
    neigh = pl.pallas_call(
        _knn2_kernel, grid=(168,),
        in_specs=[pl.BlockSpec((128, 3), lambda i: (i, 0)),
                  pl.BlockSpec((_T3, 3), lambda i: (0, 0))],
        out_specs=pl.BlockSpec((128, _KN), lambda i: (i, 0)),
        out_shape=jax.ShapeDtypeStruct((_T3, _KN), jnp.int32),
    )(bary, bary)

    if _HAS_SC:
        expand = _make_sc_gather()(packed, neigh.reshape(_T3 * _KN))
    else:
        expand = packed[neigh.reshape(-1)]

    fs = pl.pallas_call(
        _mlp_kernel, grid=(21,),
        in_specs=[pl.BlockSpec((1024 * _KN, 16), lambda i: (i, 0)),
                  pl.BlockSpec((1024, 16), lambda i: (i, 0)),
                  pl.BlockSpec((12, 128), lambda i: (0, 0)),
                  pl.BlockSpec((128, 1), lambda i: (0, 0))],
        out_specs=pl.BlockSpec((1024, 1), lambda i: (i, 0)),
        out_shape=jax.ShapeDtypeStruct((_T3, 1), jnp.float32),
    )(expand, packed, W_mlp1, w_mlp2)

    iota_t = lax.iota(jnp.int32, _T3).reshape(1, _T3)
    rank = pl.pallas_call(
        _rank_kernel, grid=(84,),
        in_specs=[pl.BlockSpec((256, 1), lambda i: (i, 0)),
                  pl.BlockSpec((1, _T3), lambda i: (0, 0)),
                  pl.BlockSpec((1, _T3), lambda i: (0, 0))],
        out_specs=pl.BlockSpec((256, 1), lambda i: (i, 0)),
        out_shape=jax.ShapeDtypeStruct((_T3, 1), jnp.int32),
    )(fs, fs.reshape(1, _T3), iota_t)

    tri_flat = packed[:, 3:12]
    out = pl.pallas_call(
        _gather_out_kernel, grid=(2,),
        in_specs=[pl.BlockSpec((_T3, 1), lambda i: (0, 0)),
                  pl.BlockSpec((_T3, 9), lambda i: (0, 0)),
                  pl.BlockSpec((1, _T3), lambda i: (0, 0))],
        out_specs=pl.BlockSpec((256, 9), lambda i: (i, 0)),
        out_shape=jax.ShapeDtypeStruct((_TARGET, 9), jnp.float32),
    )(rank, tri_flat, iota_t)
    return out.reshape(_TARGET, 3, 3)


# speedup vs baseline: 3.8398x; 3.8398x over previous
import functools

import jax
import jax.numpy as jnp
from jax import lax
from jax.experimental import pallas as pl
from jax.experimental.pallas import tpu as pltpu

try:
    from jax.experimental.pallas import tpu_sc as plsc
    _HAS_SC = True
except ImportError:
    _HAS_SC = False

_K1 = 15
_KN = 20
_M = 1536
_N = 4096
_T3 = _M * (_K1 - 1)
_TARGET = 512
_HI = lax.Precision.HIGHEST


def _sq3(g):
    p = g * g
    return (p[:, 0] + p[:, 2]) + p[:, 1]


def _sum64_rows(x):
    y = x[:, 0:8]
    for i in range(1, 8):
        y = y + x[:, 8 * i:8 * (i + 1)]
    y = y[:, 0:4] + y[:, 4:8]
    y = y[:, 0:2] + y[:, 2:4]
    return y[:, 0:1] + y[:, 1:2]


def _sum20_rows(x):
    x = jnp.concatenate([x, jnp.zeros((x.shape[0], 12), jnp.float32)], axis=1)
    n = 32
    while n > 1:
        n //= 2
        x = x[:, :n] + x[:, n:]
    return x


def _rowsum1536(s):
    acc = s[:, 0:128]
    for t in range(1, 12):
        acc = acc + s[:, 128 * t:128 * (t + 1)]
    y = acc[:, 0:8]
    for i in range(1, 16):
        y = y + acc[:, 8 * i:8 * (i + 1)]
    y = y[:, 0:4] + y[:, 4:8]
    y = y[:, 0:2] + y[:, 2:4]
    return y[:, 0:1] + y[:, 1:2]


def _topk_rows(nd, k):
    R, C = nd.shape
    cols = lax.broadcasted_iota(jnp.int32, (R, C), 1)
    idxs = []
    for _ in range(k):
        m = jnp.max(nd, axis=1, keepdims=True)
        idx = jnp.min(jnp.where(nd == m, cols, C), axis=1, keepdims=True)
        idxs.append(idx)
        nd = jnp.where(cols == idx, -jnp.inf, nd)
    return jnp.concatenate(idxs, axis=1)


def _score_kernel(adj_ref, nodes_ref, W1_ref, w2_ref, out_ref):
    h = jax.nn.relu(jnp.dot(nodes_ref[...], W1_ref[...]))
    aggT = lax.dot_general(h, adj_ref[...], (((0,), (1,)), ((), ())))
    out_ref[...] = jnp.dot(aggT.T, w2_ref[...])


def _logits_kernel(score_ref, g_ref, v_ref):
    s = score_ref[...]
    sh = s - jnp.max(s)
    v_ref[...] = sh - jnp.log(jnp.sum(jnp.exp(sh))) + g_ref[...]


def _rank_generic_kernel(vb_ref, ib_ref, vrow_ref, irow_ref, rank_ref):
    vi = vb_ref[...]
    ii = ib_ref[...]
    vr = vrow_ref[...]
    ir = irow_ref[...]
    gt = vr > vi
    eqlt = (vr == vi) & (ir < ii)
    rank_ref[...] = jnp.sum((gt | eqlt).astype(jnp.int32), axis=1, keepdims=True)


def _onehot_gather_kernel(rb_ref, rankrow_ref, table_ref, out_ref):
    onehot = (rankrow_ref[...] == rb_ref[...]).astype(jnp.float32)
    out_ref[...] = jnp.dot(onehot, table_ref[...], precision=_HI)


def _knn1_kernel(chunk_ref, gen_ref, nbr_ref):
    i = pl.program_id(0)
    chunk = chunk_ref[...]
    g = gen_ref[...]
    d = _sq3(chunk)[:, None] + _sq3(g)[None, :] - 2.0 * lax.dot_general(
        chunk, g, (((1,), (1,)), ((), ())))
    rows = lax.broadcasted_iota(jnp.int32, (512, _M), 0) + i * 512
    cols = lax.broadcasted_iota(jnp.int32, (512, _M), 1)
    nd = -jnp.where(rows == cols, jnp.inf, d)
    nbr_ref[...] = _topk_rows(nd, _K1)


def _devconv_kernel(nbrcol_ref, genrep_ref, gen_ref, Wd_ref, f_ref):
    cols = lax.broadcasted_iota(jnp.int32, (256 * _K1, _M), 1)
    onehot = (nbrcol_ref[...] == cols).astype(jnp.float32)
    gnbr = jnp.dot(onehot, gen_ref[...], precision=_HI)
    diff = gnbr - genrep_ref[...]
    z = jax.nn.relu(jnp.dot(diff, Wd_ref[...]))
    z = z.reshape(256, _K1, 64)
    se = z[:, 0, :]
    for i in range(1, _K1):
        se = jnp.maximum(se, z[:, i, :])
    f_ref[...] = jax.nn.sigmoid(_sum64_rows(se) / 64.0)


def _rowsum_kernel(nbr_ref, fcol_ref, frow_ref, rs_ref):
    nbr = nbr_ref[...]
    cols = lax.broadcasted_iota(jnp.int32, (_M, _M), 1)
    A = jnp.zeros((_M, _M), jnp.float32)
    for k in range(_K1):
        A = jnp.maximum(A, (nbr[:, k:k + 1] == cols).astype(jnp.float32))
    A = jnp.maximum(A, A.T)
    S = A * (fcol_ref[...] * frow_ref[...])
    rs_ref[...] = _rowsum1536(S)


def _packed_kernel(nbr_ref, nbrcol_ref, centerb_ref, table_ref, packed_ref):
    nbr_b = nbr_ref[...]
    cols = lax.broadcasted_iota(jnp.int32, (256 * _K1, _M), 1)
    onehot = (nbrcol_ref[...] == cols).astype(jnp.float32)
    gath = jnp.dot(onehot, table_ref[...], precision=_HI).reshape(256, _K1, 20)
    gnbr = gath[:, :, 0:3]
    f_nbr = gath[:, :, 3]
    rs_nbr = gath[:, :, 4]
    nbr_nbr = gath[:, :, 5:20]
    center = centerb_ref[...]
    v0 = center[:, 0:3]
    f_i = center[:, 3:4]
    rs_i = center[:, 4:5]
    pieces = []
    for c in range(_K1 - 1):
        jf = nbr_b[:, c:c + 1].astype(jnp.float32)
        lf = nbr_b[:, c + 1:c + 2].astype(jnp.float32)
        f_j, f_l = f_nbr[:, c:c + 1], f_nbr[:, c + 1:c + 2]
        rs_j, rs_l = rs_nbr[:, c:c + 1], rs_nbr[:, c + 1:c + 2]
        a_jl = (jnp.max((nbr_nbr[:, c, :] == lf).astype(jnp.float32),
                        axis=1, keepdims=True)
                + jnp.max((nbr_nbr[:, c + 1, :] == jf).astype(jnp.float32),
                          axis=1, keepdims=True)) > 0.0
        p_ij = 0.5 * ((f_i * f_j) / (rs_i + 1e-9) + (f_j * f_i) / (rs_j + 1e-9))
        p_il = 0.5 * ((f_i * f_l) / (rs_i + 1e-9) + (f_l * f_i) / (rs_l + 1e-9))
        p_jl_val = 0.5 * ((f_j * f_l) / (rs_j + 1e-9) + (f_l * f_j) / (rs_l + 1e-9))
        p_jl = jnp.where(a_jl, p_jl_val, 0.0)
        p_init = (p_ij * p_jl) * p_il
        v1 = gnbr[:, c, :]
        v2 = gnbr[:, c + 1, :]
        bary = ((v0 + v1) + v2) / 3.0
        row = jnp.concatenate(
            [bary, v0, v1, v2, p_init,
             jnp.zeros((256, 115), jnp.float32)], axis=1)
        pieces.append(row)
    packed_ref[...] = jnp.concatenate(pieces, axis=1)


def _knn2_kernel(chunk_ref, bary_ref, neigh_ref):
    i = pl.program_id(0)
    chunk = chunk_ref[...]
    b = bary_ref[...]
    d = _sq3(chunk)[:, None] + _sq3(b)[None, :] - 2.0 * lax.dot_general(
        chunk, b, (((1,), (1,)), ((), ())))
    rows = lax.broadcasted_iota(jnp.int32, (128, _T3), 0) + i * 128
    cols = lax.broadcasted_iota(jnp.int32, (128, _T3), 1)
    nd = -jnp.where(rows == cols, jnp.inf, d)
    neigh_ref[...] = _topk_rows(nd, _KN)


def _make_sc_gather():
    info = plsc.get_sparse_core_info()
    NW = info.num_cores * info.num_subcores
    B = _T3 * _KN
    per_w = B // NW
    chunk = 896
    mesh = plsc.VectorSubcoreMesh(core_axis_name="c", subcore_axis_name="s")

    @functools.partial(
        pl.kernel, mesh=mesh,
        out_type=jax.ShapeDtypeStruct((B, 128), jnp.float32),
        scratch_types=[
            pltpu.VMEM((chunk,), jnp.int32),
            pltpu.VMEM((chunk, 128), jnp.float32),
            pltpu.SemaphoreType.DMA,
        ],
    )
    def sc_gather(packed_hbm, idx_hbm, out_hbm, idx_v, rows_v, sem):
        wid = lax.axis_index("s") * info.num_cores + lax.axis_index("c")
        base = wid * per_w
        for k in range(per_w // chunk):
            off = base + k * chunk
            pltpu.sync_copy(idx_hbm.at[pl.ds(off, chunk)], idx_v)
            pltpu.async_copy(packed_hbm.at[idx_v], rows_v, sem).wait()
            pltpu.sync_copy(rows_v, out_hbm.at[pl.ds(off, chunk)])

    return sc_gather


def _mlp_kernel(exp_ref, packedb_ref, Wm_ref, wv_ref, out_ref):
    nb = exp_ref[...].reshape(1024, _KN, 128)
    center = packedb_ref[...]
    d_bary = nb[:, :, 0:3] - center[:, None, 0:3]
    d_tri = nb[:, :, 3:12] - center[:, None, 3:12]
    rm = jnp.concatenate([d_bary, d_tri], axis=2)
    hm = jax.nn.relu(jnp.dot(rm.reshape(1024 * _KN, 12), Wm_ref[...]))
    hm = hm.reshape(1024, _KN, 128)
    pn = nb[:, :, 12]
    w3 = hm * pn[:, :, None]
    acc = w3[:, 0, :]
    for i in range(1, _KN):
        acc = acc + w3[:, i, :]
    pooled = acc / (_sum20_rows(pn) + 1e-9)
    s = jnp.dot(pooled, wv_ref[...])
    out_ref[...] = jax.nn.sigmoid(s) * center[:, 12:13]


def kernel(original_graph_nodes, original_graph_adjacency_matrix, W_gnn1,
           w_gnn2, W_dev, W_mlp1, w_mlp2, target_number_triangles):
    nodes = original_graph_nodes
    adj = original_graph_adjacency_matrix

    u = jax.random.uniform(jax.random.key(42), (_N,), dtype=jnp.float32)
    g = -jnp.log(-jnp.log(u + 1e-9) + 1e-9)

    score = pl.pallas_call(
        _score_kernel, grid=(8,),
        in_specs=[pl.BlockSpec((512, _N), lambda i: (i, 0)),
                  pl.BlockSpec((_N, 3), lambda i: (0, 0)),
                  pl.BlockSpec((3, 64), lambda i: (0, 0)),
                  pl.BlockSpec((64, 1), lambda i: (0, 0))],
        out_specs=pl.BlockSpec((512, 1), lambda i: (i, 0)),
        out_shape=jax.ShapeDtypeStruct((_N, 1), jnp.float32),
    )(adj, nodes, W_gnn1, w_gnn2)

    iota_n_row = lax.iota(jnp.int32, _N).reshape(1, _N)
    iota_n_col = lax.broadcasted_iota(jnp.int32, (_N, 1), 0)
    iota_m_col = lax.broadcasted_iota(jnp.int32, (_M, 1), 0)
    v = pl.pallas_call(
        _logits_kernel,
        out_shape=jax.ShapeDtypeStruct((1, _N), jnp.float32),
    )(score.reshape(1, _N), g.reshape(1, _N))

    rank_n = pl.pallas_call(
        _rank_generic_kernel, grid=(8,),
        in_specs=[pl.BlockSpec((512, 1), lambda i: (i, 0)),
                  pl.BlockSpec((512, 1), lambda i: (i, 0)),
                  pl.BlockSpec((1, _N), lambda i: (0, 0)),
                  pl.BlockSpec((1, _N), lambda i: (0, 0))],
        out_specs=pl.BlockSpec((512, 1), lambda i: (i, 0)),
        out_shape=jax.ShapeDtypeStruct((_N, 1), jnp.int32),
    )(v.reshape(_N, 1), iota_n_col, v, iota_n_row)

    gen = pl.pallas_call(
        _onehot_gather_kernel, grid=(3,),
        in_specs=[pl.BlockSpec((512, 1), lambda i: (i, 0)),
                  pl.BlockSpec((1, _N), lambda i: (0, 0)),
                  pl.BlockSpec((_N, 3), lambda i: (0, 0))],
        out_specs=pl.BlockSpec((512, 3), lambda i: (i, 0)),
        out_shape=jax.ShapeDtypeStruct((_M, 3), jnp.float32),
    )(iota_m_col, rank_n.reshape(1, _N), nodes)

    nbr = pl.pallas_call(
        _knn1_kernel, grid=(3,),
        in_specs=[pl.BlockSpec((512, 3), lambda i: (i, 0)),
                  pl.BlockSpec((_M, 3), lambda i: (0, 0))],
        out_specs=pl.BlockSpec((512, _K1), lambda i: (i, 0)),
        out_shape=jax.ShapeDtypeStruct((_M, _K1), jnp.int32),
    )(gen, gen)

    nbr_col = nbr.reshape(_M * _K1, 1)
    gen_rep = jnp.repeat(gen, _K1, axis=0)
    f = pl.pallas_call(
        _devconv_kernel, grid=(6,),
        in_specs=[pl.BlockSpec((256 * _K1, 1), lambda i: (i, 0)),
                  pl.BlockSpec((256 * _K1, 3), lambda i: (i, 0)),
                  pl.BlockSpec((_M, 3), lambda i: (0, 0)),
                  pl.BlockSpec((3, 64), lambda i: (0, 0))],
        out_specs=pl.BlockSpec((256, 1), lambda i: (i, 0)),
        out_shape=jax.ShapeDtypeStruct((_M, 1), jnp.float32),
    )(nbr_col, gen_rep, gen, W_dev)

    rs = pl.pallas_call(
        _rowsum_kernel,
        out_shape=jax.ShapeDtypeStruct((_M, 1), jnp.float32),
    )(nbr, f, f.reshape(1, _M))

    table = jnp.concatenate([gen, f, rs, nbr.astype(jnp.float32)], axis=1)
    centerb = jnp.concatenate([gen, f, rs], axis=1)
    packed_rows = pl.pallas_call(
        _packed_kernel, grid=(6,),
        in_specs=[pl.BlockSpec((256, _K1), lambda i: (i, 0)),
                  pl.BlockSpec((256 * _K1, 1), lambda i: (i, 0)),
                  pl.BlockSpec((256, 5), lambda i: (i, 0)),
                  pl.BlockSpec((_M, 20), lambda i: (0, 0))],
        out_specs=pl.BlockSpec((256, (_K1 - 1) * 128), lambda i: (i, 0)),
        out_shape=jax.ShapeDtypeStruct((_M, (_K1 - 1) * 128), jnp.float32),
    )(nbr, nbr_col, centerb, table)
    packed = packed_rows.reshape(_T3, 128)

    bary = packed[:, 0:3]
    neigh = pl.pallas_call(
        _knn2_kernel, grid=(168,),
        in_specs=[pl.BlockSpec((128, 3), lambda i: (i, 0)),
                  pl.BlockSpec((_T3, 3), lambda i: (0, 0))],
        out_specs=pl.BlockSpec((128, _KN), lambda i: (i, 0)),
        out_shape=jax.ShapeDtypeStruct((_T3, _KN), jnp.int32),
    )(bary, bary)

    if _HAS_SC:
        expand = _make_sc_gather()(packed, neigh.reshape(_T3 * _KN))
    else:
        expand = packed[neigh.reshape(-1)]

    fs = pl.pallas_call(
        _mlp_kernel, grid=(21,),
        in_specs=[pl.BlockSpec((1024 * _KN, 128), lambda i: (i, 0)),
                  pl.BlockSpec((1024, 128), lambda i: (i, 0)),
                  pl.BlockSpec((12, 128), lambda i: (0, 0)),
                  pl.BlockSpec((128, 1), lambda i: (0, 0))],
        out_specs=pl.BlockSpec((1024, 1), lambda i: (i, 0)),
        out_shape=jax.ShapeDtypeStruct((_T3, 1), jnp.float32),
    )(expand, packed, W_mlp1, w_mlp2)

    iota_t_row = lax.iota(jnp.int32, _T3).reshape(1, _T3)
    iota_t_col = lax.broadcasted_iota(jnp.int32, (_T3, 1), 0)
    rank_t = pl.pallas_call(
        _rank_generic_kernel, grid=(84,),
        in_specs=[pl.BlockSpec((256, 1), lambda i: (i, 0)),
                  pl.BlockSpec((256, 1), lambda i: (i, 0)),
                  pl.BlockSpec((1, _T3), lambda i: (0, 0)),
                  pl.BlockSpec((1, _T3), lambda i: (0, 0))],
        out_specs=pl.BlockSpec((256, 1), lambda i: (i, 0)),
        out_shape=jax.ShapeDtypeStruct((_T3, 1), jnp.int32),
    )(fs, iota_t_col, fs.reshape(1, _T3), iota_t_row)

    iota_o_col = lax.broadcasted_iota(jnp.int32, (_TARGET, 1), 0)
    tri_flat = packed[:, 3:12]
    out = pl.pallas_call(
        _onehot_gather_kernel, grid=(2,),
        in_specs=[pl.BlockSpec((256, 1), lambda i: (i, 0)),
                  pl.BlockSpec((1, _T3), lambda i: (0, 0)),
                  pl.BlockSpec((_T3, 9), lambda i: (0, 0))],
        out_specs=pl.BlockSpec((256, 9), lambda i: (i, 0)),
        out_shape=jax.ShapeDtypeStruct((_TARGET, 9), jnp.float32),
    )(iota_o_col, rank_t.reshape(1, _T3), tri_flat)
    return out.reshape(_TARGET, 3, 3)
